# Initial kernel scaffold; baseline (speedup 1.0000x reference)
import functools
import math

import jax
import jax.numpy as jnp
import numpy as np
from jax import lax
from jax.experimental import pallas as pl
from jax.experimental.pallas import tpu as pltpu
from jax.experimental.pallas import tpu_sc as plsc

D_MODEL = 128
SEQ = 200
BATCH = 1024
SCALE = math.sqrt(D_MODEL)

NC = 2
NS = 16
NW = NC * NS

TOTAL_ROWS = BATCH * SEQ
ROWS_PER_W = TOTAL_ROWS // NW
CHUNK = 128
CHUNKS_PER_W = ROWS_PER_W // CHUNK
LANES = 16
VECS_PER_ROW = D_MODEL // LANES


def _make_pe_np():
    position = np.arange(SEQ, dtype=np.float32)[:, None]
    div_term = np.exp(
        np.arange(0, D_MODEL, 2, dtype=np.float32) * (-math.log(10000.0) / D_MODEL)
    )
    pe = np.zeros((SEQ, D_MODEL), dtype=np.float32)
    pe[:, 0::2] = np.sin(position * div_term)
    pe[:, 1::2] = np.cos(position * div_term)
    return pe


_PE = jnp.asarray(_make_pe_np())

_mesh = plsc.VectorSubcoreMesh(core_axis_name="c", subcore_axis_name="s")


@functools.partial(
    pl.kernel,
    mesh=_mesh,
    out_type=jax.ShapeDtypeStruct((TOTAL_ROWS, D_MODEL), jnp.float32),
    scratch_types=[
        pltpu.VMEM((CHUNK,), jnp.int32),
        pltpu.VMEM((CHUNK, D_MODEL), jnp.float32),
        pltpu.VMEM((SEQ, D_MODEL), jnp.float32),
        pltpu.SemaphoreType.DMA,
    ],
)
def _sc_embed(table_hbm, idx_hbm, pe_hbm, out_hbm, idx_v, rows_v, pe_v, sem):
    wid = lax.axis_index("s") * NC + lax.axis_index("c")
    base = wid * ROWS_PER_W
    pltpu.sync_copy(pe_hbm, pe_v)

    def chunk_body(c, _):
        off = base + c * CHUNK
        pltpu.sync_copy(idx_hbm.at[pl.ds(off, CHUNK)], idx_v)
        pltpu.async_copy(table_hbm.at[idx_v], rows_v, sem).wait()

        def row_body(i, _):
            pos = lax.rem(c * CHUNK + i, SEQ)
            for j in range(VECS_PER_ROW):
                sl = pl.ds(j * LANES, LANES)
                rows_v[i, sl] = rows_v[i, sl] * SCALE + pe_v[pos, sl]
            return 0

        lax.fori_loop(0, CHUNK, row_body, 0)
        pltpu.sync_copy(rows_v, out_hbm.at[pl.ds(off, CHUNK)])
        return 0

    lax.fori_loop(0, CHUNKS_PER_W, chunk_body, 0)


def kernel(x, table):
    idx = x.reshape(TOTAL_ROWS).astype(jnp.int32)
    out = _sc_embed(table, idx, _PE)
    return out.reshape(BATCH, SEQ, D_MODEL)


# speedup vs baseline: 1.9482x; 1.9482x over previous
import functools
import math

import jax
import jax.numpy as jnp
import numpy as np
from jax import lax
from jax.experimental import pallas as pl
from jax.experimental.pallas import tpu as pltpu
from jax.experimental.pallas import tpu_sc as plsc

D_MODEL = 128
SEQ = 200
BATCH = 1024
SCALE = math.sqrt(D_MODEL)

NC = 2
NS = 16
NW = NC * NS

TOTAL_ROWS = BATCH * SEQ
ROWS_PER_W = TOTAL_ROWS // NW
CHUNK = 128
CHUNKS_PER_W = ROWS_PER_W // CHUNK
LANES = 16
VECS_PER_ROW = D_MODEL // LANES


def _make_pe_np():
    position = np.arange(SEQ, dtype=np.float32)[:, None]
    div_term = np.exp(
        np.arange(0, D_MODEL, 2, dtype=np.float32) * (-math.log(10000.0) / D_MODEL)
    )
    pe = np.zeros((SEQ, D_MODEL), dtype=np.float32)
    pe[:, 0::2] = np.sin(position * div_term)
    pe[:, 1::2] = np.cos(position * div_term)
    return pe


_PE_NP = _make_pe_np()

_mesh = plsc.VectorSubcoreMesh(core_axis_name="c", subcore_axis_name="s")


@functools.partial(
    pl.kernel,
    mesh=_mesh,
    out_type=jax.ShapeDtypeStruct((TOTAL_ROWS, D_MODEL), jnp.float32),
    scratch_types=[
        pltpu.VMEM((CHUNK,), jnp.int32),
        pltpu.VMEM((CHUNK, D_MODEL), jnp.float32),
        pltpu.VMEM((SEQ, D_MODEL), jnp.float32),
        pltpu.SemaphoreType.DMA,
    ],
)
def _sc_embed(table_hbm, idx_hbm, pe_hbm, out_hbm, idx_v, rows_v, pe_v, sem):
    wid = lax.axis_index("s") * NC + lax.axis_index("c")
    base = wid * ROWS_PER_W
    pltpu.sync_copy(pe_hbm, pe_v)

    def chunk_body(c, _):
        off = base + c * CHUNK
        pltpu.sync_copy(idx_hbm.at[pl.ds(off, CHUNK)], idx_v)
        pltpu.async_copy(table_hbm.at[idx_v], rows_v, sem).wait()

        def row_body(i, _):
            pos = lax.rem(c * CHUNK + i, SEQ)
            for j in range(VECS_PER_ROW):
                sl = pl.ds(j * LANES, LANES)
                rows_v[i, sl] = rows_v[i, sl] * SCALE + pe_v[pos, sl]
            return 0

        lax.fori_loop(0, CHUNK, row_body, 0)
        pltpu.sync_copy(rows_v, out_hbm.at[pl.ds(off, CHUNK)])
        return 0

    lax.fori_loop(0, CHUNKS_PER_W, chunk_body, 0)


def kernel(x, table):
    idx = x.reshape(TOTAL_ROWS).astype(jnp.int32)
    out = _sc_embed(table, idx, jnp.asarray(_PE_NP))
    return out.reshape(BATCH, SEQ, D_MODEL)


# speedup vs baseline: 2.5624x; 1.3153x over previous
import functools
import math

import jax
import jax.numpy as jnp
import numpy as np
from jax import lax
from jax.experimental import pallas as pl
from jax.experimental.pallas import tpu as pltpu
from jax.experimental.pallas import tpu_sc as plsc

D_MODEL = 128
SEQ = 200
BATCH = 1024
SCALE = math.sqrt(D_MODEL)

NC = 2
NS = 16
NW = NC * NS

TOTAL_ROWS = BATCH * SEQ
ROWS_PER_W = TOTAL_ROWS // NW
CHUNK = 128
NCHUNK = ROWS_PER_W // CHUNK
LANES = 16
VECS_PER_ROW = D_MODEL // LANES


def _make_pe_np():
    position = np.arange(SEQ, dtype=np.float32)[:, None]
    div_term = np.exp(
        np.arange(0, D_MODEL, 2, dtype=np.float32) * (-math.log(10000.0) / D_MODEL)
    )
    pe = np.zeros((SEQ, D_MODEL), dtype=np.float32)
    pe[:, 0::2] = np.sin(position * div_term)
    pe[:, 1::2] = np.cos(position * div_term)
    return pe


_PE_NP = _make_pe_np()

_mesh = plsc.VectorSubcoreMesh(core_axis_name="c", subcore_axis_name="s")


@functools.partial(
    pl.kernel,
    mesh=_mesh,
    out_type=jax.ShapeDtypeStruct((TOTAL_ROWS, D_MODEL), jnp.float32),
    scratch_types=[
        pltpu.VMEM((ROWS_PER_W,), jnp.int32),
        pltpu.VMEM((CHUNK, D_MODEL), jnp.float32),
        pltpu.VMEM((CHUNK, D_MODEL), jnp.float32),
        pltpu.VMEM((SEQ, D_MODEL), jnp.float32),
        pltpu.SemaphoreType.DMA,
        pltpu.SemaphoreType.DMA,
        pltpu.SemaphoreType.DMA,
        pltpu.SemaphoreType.DMA,
    ],
)
def _sc_embed(table_hbm, idx_hbm, pe_hbm, out_hbm,
              idx_all, rows0, rows1, pe_v, g0, g1, w0, w1):
    wid = lax.axis_index("s") * NC + lax.axis_index("c")
    base = wid * ROWS_PER_W
    rows = (rows0, rows1)
    gsem = (g0, g1)
    wsem = (w0, w1)

    pltpu.sync_copy(pe_hbm, pe_v)
    pltpu.sync_copy(idx_hbm.at[pl.ds(base, ROWS_PER_W)], idx_all)

    def gather(c, b):
        pltpu.make_async_copy(
            table_hbm.at[idx_all.at[pl.ds(c * CHUNK, CHUNK)]], rows[b], gsem[b]
        ).start()

    def gather_wait(b):
        pltpu.make_async_copy(
            table_hbm.at[idx_all.at[pl.ds(0, CHUNK)]], rows[b], gsem[b]
        ).wait()

    def wb_start(c, b):
        pltpu.make_async_copy(
            rows[b], out_hbm.at[pl.ds(base + c * CHUNK, CHUNK)], wsem[b]
        ).start()

    def wb_wait(b):
        pltpu.make_async_copy(
            rows[b], out_hbm.at[pl.ds(base, CHUNK)], wsem[b]
        ).wait()

    gather(0, 0)

    def step(c, b, rv):
        @pl.when(c >= 1)
        def _():
            wb_wait(1 - b)

        @pl.when(c <= NCHUNK - 2)
        def _():
            gather(c + 1, 1 - b)

        gather_wait(b)

        def row_body(i, _):
            pos = lax.rem(c * CHUNK + i, SEQ)
            for j in range(VECS_PER_ROW):
                sl = pl.ds(j * LANES, LANES)
                rv[i, sl] = rv[i, sl] * SCALE + pe_v[pos, sl]
            return 0

        lax.fori_loop(0, CHUNK, row_body, 0)
        wb_start(c, b)

    def pair_body(c2, _):
        step(2 * c2, 0, rows0)
        step(2 * c2 + 1, 1, rows1)
        return 0

    lax.fori_loop(0, NCHUNK // 2, pair_body, 0)
    wb_wait(1)


def kernel(x, table):
    idx = x.reshape(TOTAL_ROWS).astype(jnp.int32)
    out = _sc_embed(table, idx, jnp.asarray(_PE_NP))
    return out.reshape(BATCH, SEQ, D_MODEL)


# speedup vs baseline: 7.5290x; 2.9382x over previous
import functools
import math

import jax
import jax.numpy as jnp
import numpy as np
from jax import lax
from jax.experimental import pallas as pl
from jax.experimental.pallas import tpu as pltpu
from jax.experimental.pallas import tpu_sc as plsc

D_MODEL = 128
SEQ = 200
BATCH = 1024
SCALE = math.sqrt(D_MODEL)

NC = 2
NS = 16
NW = NC * NS

TOTAL_ROWS = BATCH * SEQ
ROWS_PER_W = TOTAL_ROWS // NW
CHUNK = 128
NCHUNK = ROWS_PER_W // CHUNK
LANES = 16
VECS_PER_ROW = D_MODEL // LANES
PE_EXP = SEQ + CHUNK
NBUF = 4


def _make_pe_np():
    position = np.arange(SEQ, dtype=np.float32)[:, None]
    div_term = np.exp(
        np.arange(0, D_MODEL, 2, dtype=np.float32) * (-math.log(10000.0) / D_MODEL)
    )
    pe = np.zeros((SEQ, D_MODEL), dtype=np.float32)
    pe[:, 0::2] = np.sin(position * div_term)
    pe[:, 1::2] = np.cos(position * div_term)
    pe_exp = np.concatenate([pe, pe[: PE_EXP - SEQ]], axis=0)
    return np.ascontiguousarray(pe_exp / np.float32(SCALE), dtype=np.float32)


_PE_DIV_NP = _make_pe_np()

_mesh = plsc.VectorSubcoreMesh(core_axis_name="c", subcore_axis_name="s")


@functools.partial(
    pl.kernel,
    mesh=_mesh,
    out_type=jax.ShapeDtypeStruct((TOTAL_ROWS, D_MODEL), jnp.float32),
    scratch_types=[
        pltpu.VMEM((ROWS_PER_W,), jnp.int32),
        pltpu.VMEM((CHUNK, D_MODEL), jnp.float32),
        pltpu.VMEM((CHUNK, D_MODEL), jnp.float32),
        pltpu.VMEM((CHUNK, D_MODEL), jnp.float32),
        pltpu.VMEM((CHUNK, D_MODEL), jnp.float32),
        pltpu.VMEM_SHARED((PE_EXP, D_MODEL), jnp.float32),
        pltpu.SemaphoreType.DMA((NBUF,)),
        pltpu.SemaphoreType.DMA((NBUF,)),
        pltpu.SemaphoreType.DMA((NBUF,)),
    ],
)
def _sc_embed(table_hbm, idx_hbm, pe_hbm, out_hbm,
              idx_all, rb0, rb1, rb2, rb3, pe_sh, pf_sem, ga_sem, wb_sem):
    wid = lax.axis_index("s") * NC + lax.axis_index("c")
    base = wid * ROWS_PER_W
    rows = (rb0, rb1, rb2, rb3)

    @pl.when(lax.axis_index("s") == 0)
    def _():
        pltpu.sync_copy(pe_hbm, pe_sh)

    pltpu.sync_copy(idx_hbm.at[pl.ds(base, ROWS_PER_W)], idx_all)
    plsc.subcore_barrier()

    def pf_start(c, m):
        p0 = lax.rem(c * CHUNK, SEQ)
        pltpu.async_copy(pe_sh.at[pl.ds(p0, CHUNK)], rows[m], pf_sem.at[m])

    def pf_wait(m):
        pltpu.make_async_copy(pe_sh.at[pl.ds(0, CHUNK)], rows[m],
                              pf_sem.at[m]).wait()

    def ga_start(c, m):
        pltpu.async_copy(
            table_hbm.at[idx_all.at[pl.ds(c * CHUNK, CHUNK)]], rows[m],
            ga_sem.at[m], add=True)

    def ga_wait(m):
        pltpu.make_async_copy(
            table_hbm.at[idx_all.at[pl.ds(0, CHUNK)]], rows[m],
            ga_sem.at[m]).wait()

    def wb_start(c, m):
        pltpu.async_copy(rows[m], out_hbm.at[pl.ds(base + c * CHUNK, CHUNK)],
                         wb_sem.at[m])

    def wb_wait(m):
        pltpu.make_async_copy(rows[m], out_hbm.at[pl.ds(base, CHUNK)],
                              wb_sem.at[m]).wait()

    def scale_pass(rv):
        def row_body(i, _):
            for j in range(VECS_PER_ROW):
                sl = pl.ds(j * LANES, LANES)
                rv[i, sl] = rv[i, sl] * SCALE
            return 0
        lax.fori_loop(0, CHUNK, row_body, 0)

    def step(c, r, first=False, pf_c2=True, ga_c1=True):
        if not first:
            wb_wait((r + 2) % NBUF)
        if pf_c2:
            pf_start(c + 2, (r + 2) % NBUF)
        if ga_c1:
            pf_wait((r + 1) % NBUF)
            ga_start(c + 1, (r + 1) % NBUF)
        ga_wait(r)
        scale_pass(rows[r])
        wb_start(c, r)

    pf_start(0, 0)
    pf_start(1, 1)
    pf_wait(0)
    ga_start(0, 0)

    step(0, 0, first=True)
    step(1, 1, first=True)

    def group(c4, _):
        c = 2 + 4 * c4
        step(c + 0, 2)
        step(c + 1, 3)
        step(c + 2, 0)
        step(c + 3, 1)
        return 0

    lax.fori_loop(0, (NCHUNK - 6) // NBUF, group, 0)

    step(46, 2)
    step(47, 3)
    step(48, 0, pf_c2=False)
    step(49, 1, pf_c2=False, ga_c1=False)
    wb_wait(0)
    wb_wait(1)


def kernel(x, table):
    idx = x.reshape(TOTAL_ROWS).astype(jnp.int32)
    out = _sc_embed(table, idx, jnp.asarray(_PE_DIV_NP))
    return out.reshape(BATCH, SEQ, D_MODEL)
